# Initial kernel scaffold; baseline (speedup 1.0000x reference)
import functools

import jax
import jax.numpy as jnp
from jax import lax
from jax.experimental import pallas as pl
from jax.experimental.pallas import tpu as pltpu
from jax.experimental.pallas import tpu_sc as plsc

N = 10000
D = 128
DH = 64
E = 320000
E2 = 2 * E
NS = 16
L = 16

EB = 128
ROWS_PER_TILE = 313
E2P = NS * ROWS_PER_TILE * EB
NPADROWS = 16
NT = N + NPADROWS
NDEG = 10240
RT = N // NS
RB = 125
MAGIC = jnp.int32(0x5F3759DF)


def _rsqrt16(x):
    xi = plsc.bitcast(x, jnp.int32)
    y = plsc.bitcast(MAGIC - lax.shift_right_logical(xi, 1), jnp.float32)
    half = jnp.full((L,), 0.5, jnp.float32)
    three_half = jnp.full((L,), 1.5, jnp.float32)
    hx = half * x
    for _ in range(3):
        y = y * (three_half - hx * y * y)
    return y


def _splat(dis_l, idx_scalar):
    return plsc.load_gather(dis_l, [jnp.full((L,), idx_scalar, jnp.int32)])


def _body(row_hbm, col_hbm, x_hbm, out_hbm, x1_hbm,
          deg_sp, dis_sp, y_sp, acc_sp,
          dis_l, rowb, colb, stage, rowsA, rowsB, rowsC, zbuf,
          ones_b, sbufA, sem):
    c = lax.axis_index("c")
    s = lax.axis_index("s")
    r0 = s * RT
    xbase = c * N

    zero16 = jnp.zeros((L,), jnp.float32)
    for i in range(EB // L):
        ones_b[pl.ds(L * i, L)] = jnp.full((L,), 1.0, jnp.float32)
    for i in range(640 // L):
        sbufA[pl.ds(L * i, L)] = zero16

    def _zrow(r, _):
        for j in range(DH // L):
            zbuf[r, pl.ds(L * j, L)] = zero16
        return 0
    lax.fori_loop(0, RB, _zrow, 0)

    pltpu.sync_copy(sbufA, deg_sp.at[pl.ds(s * 640, 640)])
    for kk in range(RT // RB):
        pltpu.sync_copy(zbuf, acc_sp.at[pl.ds(r0 + kk * RB, RB)])
    pltpu.sync_copy(zbuf.at[pl.ds(0, 1)], y_sp.at[pl.ds(N + s, 1)])
    pltpu.sync_copy(zbuf.at[pl.ds(0, 1)], acc_sp.at[pl.ds(N + s, 1)])
    plsc.subcore_barrier()

    def _deg_chunk(k, _):
        ir = s * ROWS_PER_TILE + k
        pltpu.sync_copy(col_hbm.at[pl.ds(ir, 1)], colb)
        pltpu.sync_copy(ones_b, deg_sp.at[colb.at[0]], add=True)
        return 0
    lax.fori_loop(0, ROWS_PER_TILE, _deg_chunk, 0)
    plsc.subcore_barrier()

    pltpu.sync_copy(deg_sp.at[pl.ds(s * 640, 640)], sbufA)

    def _rs(i, _):
        sbufA[pl.ds(L * i, L)] = _rsqrt16(sbufA[pl.ds(L * i, L)])
        return 0
    lax.fori_loop(0, 640 // L, _rs, 0)
    pltpu.sync_copy(sbufA, dis_sp.at[pl.ds(s * 640, 640)])
    plsc.subcore_barrier()
    pltpu.sync_copy(dis_sp, dis_l)

    def _scale_store(kk_base, src_ref, alt_ref, third_ref, mode):
        def body(r, _):
            d = _splat(dis_l, kk_base + r)
            for j in range(DH // L):
                sl = pl.ds(L * j, L)
                v = src_ref[r, sl] * d
                if mode == 1:
                    src_ref[r, sl] = v
                    alt_ref[r, sl] = v * d
                elif mode == 2:
                    alt_ref[r, sl] = (alt_ref[r, sl] + third_ref[r, sl] + v) \
                        * jnp.full((L,), 1.0 / 3.0, jnp.float32)
                else:
                    src_ref[r, sl] = v
            return 0
        lax.fori_loop(0, RB, body, 0)

    for kk in range(RT // RB):
        rb = r0 + kk * RB
        pltpu.sync_copy(x_hbm.at[pl.ds(xbase + rb, RB)], rowsA)
        _scale_store(rb, rowsA, None, None, 0)
        pltpu.sync_copy(rowsA, y_sp.at[pl.ds(rb, RB)])
    plsc.subcore_barrier()

    def _spmm():
        def chunk(k, _):
            ir = s * ROWS_PER_TILE + k
            pltpu.sync_copy(row_hbm.at[pl.ds(ir, 1)], rowb)
            pltpu.sync_copy(col_hbm.at[pl.ds(ir, 1)], colb)
            pltpu.async_copy(y_sp.at[rowb.at[0]], stage, sem).wait()
            pltpu.sync_copy(stage, acc_sp.at[colb.at[0]], add=True)
            return 0
        lax.fori_loop(0, ROWS_PER_TILE, chunk, 0)

    _spmm()
    plsc.subcore_barrier()

    for kk in range(RT // RB):
        rb = r0 + kk * RB
        pltpu.sync_copy(acc_sp.at[pl.ds(rb, RB)], rowsB)
        _scale_store(rb, rowsB, rowsA, None, 1)
        pltpu.sync_copy(rowsB, x1_hbm.at[pl.ds(xbase + rb, RB)])
        pltpu.sync_copy(rowsA, y_sp.at[pl.ds(rb, RB)])
        pltpu.sync_copy(zbuf, acc_sp.at[pl.ds(rb, RB)])
    plsc.subcore_barrier()

    _spmm()
    plsc.subcore_barrier()

    for kk in range(RT // RB):
        rb = r0 + kk * RB
        pltpu.sync_copy(acc_sp.at[pl.ds(rb, RB)], rowsB)
        pltpu.sync_copy(x_hbm.at[pl.ds(xbase + rb, RB)], rowsA)
        pltpu.sync_copy(x1_hbm.at[pl.ds(xbase + rb, RB)], rowsC)
        _scale_store(rb, rowsB, rowsA, rowsC, 2)
        pltpu.sync_copy(rowsA, out_hbm.at[pl.ds(xbase + rb, RB)])


@jax.jit
def _gcn(row2d, col2d, xh):
    mesh = plsc.VectorSubcoreMesh(core_axis_name="c", subcore_axis_name="s")
    f = pl.kernel(
        _body,
        out_type=(
            jax.ShapeDtypeStruct((2 * N, DH), jnp.float32),
            jax.ShapeDtypeStruct((2 * N, DH), jnp.float32),
        ),
        mesh=mesh,
        scratch_types=[
            pltpu.VMEM_SHARED((NDEG,), jnp.float32),
            pltpu.VMEM_SHARED((NDEG,), jnp.float32),
            pltpu.VMEM_SHARED((NT, DH), jnp.float32),
            pltpu.VMEM_SHARED((NT, DH), jnp.float32),
            pltpu.VMEM((NDEG,), jnp.float32),
            pltpu.VMEM((1, EB), jnp.int32),
            pltpu.VMEM((1, EB), jnp.int32),
            pltpu.VMEM((EB, DH), jnp.float32),
            pltpu.VMEM((RB, DH), jnp.float32),
            pltpu.VMEM((RB, DH), jnp.float32),
            pltpu.VMEM((RB, DH), jnp.float32),
            pltpu.VMEM((RB, DH), jnp.float32),
            pltpu.VMEM((EB,), jnp.float32),
            pltpu.VMEM((640,), jnp.float32),
            pltpu.SemaphoreType.DMA,
        ],
    )
    return f(row2d, col2d, xh)


def kernel(x, adj_t):
    a = adj_t.astype(jnp.int32)
    row = jnp.concatenate([a[0], a[1]])
    col = jnp.concatenate([a[1], a[0]])
    npad = E2P - E2
    pad = N + (jnp.arange(npad, dtype=jnp.int32) % NPADROWS)
    row2d = jnp.concatenate([row, pad]).reshape(E2P // EB, EB)
    col2d = jnp.concatenate([col, pad]).reshape(E2P // EB, EB)
    xh = x.reshape(N, 2, DH).transpose(1, 0, 2).reshape(2 * N, DH)
    out2, _ = _gcn(row2d, col2d, xh)
    return out2.reshape(2, N, DH).transpose(1, 0, 2).reshape(N, D)


# speedup vs baseline: 22.3673x; 22.3673x over previous
import jax
import jax.numpy as jnp
from jax import lax
from jax.experimental import pallas as pl
from jax.experimental.pallas import tpu as pltpu
from jax.experimental.pallas import tpu_sc as plsc

N = 10000
D = 128
DH = 64
E = 320000
E2 = 2 * E
NS = 16
L = 16

EB = 128
IRT = 320
E2P = NS * IRT * EB
NT = 10240
NPADROWS = NT - N
RT = NT // NS
RB = 64
MAGIC = 0x5F3759DF


def _rsqrt16(x):
    xi = lax.bitcast_convert_type(x, jnp.int32)
    magic = jnp.full((L,), MAGIC, jnp.int32)
    y = lax.bitcast_convert_type(
        magic - lax.shift_right_logical(xi, 1), jnp.float32)
    half = jnp.full((L,), 0.5, jnp.float32)
    three_half = jnp.full((L,), 1.5, jnp.float32)
    hx = half * x
    for _ in range(3):
        y = y * (three_half - hx * y * y)
    return y


def _splat(dis_l, idx_scalar):
    return plsc.load_gather(dis_l, [jnp.full((L,), idx_scalar, jnp.int32)])


def _body(row_hbm, col_hbm, x_hbm, out_hbm, x1_hbm,
          deg_sp, dis_sp, y_sp, acc_sp,
          dis_l, rowb, colb, stage, rowsA, rowsB, rowsC, zbuf,
          ones_b, sbufA, sem):
    c = lax.axis_index("c")
    s = lax.axis_index("s")
    r0 = s * RT
    xbase = c * NT

    zero16 = jnp.zeros((L,), jnp.float32)
    for i in range(EB // L):
        ones_b[pl.ds(L * i, L)] = jnp.full((L,), 1.0, jnp.float32)
    for i in range(RT // L):
        sbufA[pl.ds(L * i, L)] = zero16

    def _zrow(r, _):
        for j in range(DH // L):
            zbuf[r, pl.ds(L * j, L)] = zero16
        return 0
    lax.fori_loop(0, RB, _zrow, 0)

    pltpu.sync_copy(sbufA, deg_sp.at[pl.ds(s * RT, RT)])
    for kk in range(RT // RB):
        pltpu.sync_copy(zbuf, acc_sp.at[pl.ds(r0 + kk * RB, RB)])
    plsc.subcore_barrier()

    def _deg_chunk(k, _):
        ir8 = s * IRT + k * 8
        pltpu.sync_copy(col_hbm.at[pl.ds(ir8, 8)], colb)
        for j in range(8):
            pltpu.sync_copy(ones_b, deg_sp.at[colb.at[j]], add=True)
        return 0
    lax.fori_loop(0, IRT // 8, _deg_chunk, 0)
    plsc.subcore_barrier()

    pltpu.sync_copy(deg_sp.at[pl.ds(s * RT, RT)], sbufA)

    def _rs(i, _):
        sbufA[pl.ds(L * i, L)] = _rsqrt16(sbufA[pl.ds(L * i, L)])
        return 0
    lax.fori_loop(0, RT // L, _rs, 0)
    pltpu.sync_copy(sbufA, dis_sp.at[pl.ds(s * RT, RT)])
    plsc.subcore_barrier()
    pltpu.sync_copy(dis_sp, dis_l)

    def _scale_store(kk_base, src_ref, alt_ref, third_ref, mode):
        def body(r, _):
            d = _splat(dis_l, kk_base + r)
            for j in range(DH // L):
                sl = pl.ds(L * j, L)
                v = src_ref[r, sl] * d
                if mode == 1:
                    src_ref[r, sl] = v
                    alt_ref[r, sl] = v * d
                elif mode == 2:
                    alt_ref[r, sl] = (alt_ref[r, sl] + third_ref[r, sl] + v) \
                        * jnp.full((L,), 1.0 / 3.0, jnp.float32)
                else:
                    src_ref[r, sl] = v
            return 0
        lax.fori_loop(0, RB, body, 0)

    for kk in range(RT // RB):
        rb = r0 + kk * RB
        pltpu.sync_copy(x_hbm.at[pl.ds(xbase + rb, RB)], rowsA)
        _scale_store(rb, rowsA, None, None, 0)
        pltpu.sync_copy(rowsA, y_sp.at[pl.ds(rb, RB)])
    plsc.subcore_barrier()

    def _spmm():
        def chunk(k, _):
            ir8 = s * IRT + k * 8
            pltpu.sync_copy(row_hbm.at[pl.ds(ir8, 8)], rowb)
            pltpu.sync_copy(col_hbm.at[pl.ds(ir8, 8)], colb)
            for j in range(8):
                pltpu.async_copy(y_sp.at[rowb.at[j]], stage, sem).wait()
                pltpu.sync_copy(stage, acc_sp.at[colb.at[j]], add=True)
            return 0
        lax.fori_loop(0, IRT // 8, chunk, 0)

    _spmm()
    plsc.subcore_barrier()

    for kk in range(RT // RB):
        rb = r0 + kk * RB
        pltpu.sync_copy(acc_sp.at[pl.ds(rb, RB)], rowsB)
        _scale_store(rb, rowsB, rowsA, None, 1)
        pltpu.sync_copy(rowsB, x1_hbm.at[pl.ds(xbase + rb, RB)])
        pltpu.sync_copy(rowsA, y_sp.at[pl.ds(rb, RB)])
        pltpu.sync_copy(zbuf, acc_sp.at[pl.ds(rb, RB)])
    plsc.subcore_barrier()

    _spmm()
    plsc.subcore_barrier()

    for kk in range(RT // RB):
        rb = r0 + kk * RB
        pltpu.sync_copy(acc_sp.at[pl.ds(rb, RB)], rowsB)
        pltpu.sync_copy(x_hbm.at[pl.ds(xbase + rb, RB)], rowsA)
        pltpu.sync_copy(x1_hbm.at[pl.ds(xbase + rb, RB)], rowsC)
        _scale_store(rb, rowsB, rowsA, rowsC, 2)
        pltpu.sync_copy(rowsA, out_hbm.at[pl.ds(xbase + rb, RB)])


@jax.jit
def _gcn(row2d, col2d, xh):
    mesh = plsc.VectorSubcoreMesh(core_axis_name="c", subcore_axis_name="s")
    f = pl.kernel(
        _body,
        out_type=(
            jax.ShapeDtypeStruct((2 * NT, DH), jnp.float32),
            jax.ShapeDtypeStruct((2 * NT, DH), jnp.float32),
        ),
        mesh=mesh,
        compiler_params=pltpu.CompilerParams(
            needs_layout_passes=False, use_tc_tiling_on_sc=False),
        scratch_types=[
            pltpu.VMEM_SHARED((NT,), jnp.float32),
            pltpu.VMEM_SHARED((NT,), jnp.float32),
            pltpu.VMEM_SHARED((NT, DH), jnp.float32),
            pltpu.VMEM_SHARED((NT, DH), jnp.float32),
            pltpu.VMEM((NT,), jnp.float32),
            pltpu.VMEM((8, EB), jnp.int32),
            pltpu.VMEM((8, EB), jnp.int32),
            pltpu.VMEM((EB, DH), jnp.float32),
            pltpu.VMEM((RB, DH), jnp.float32),
            pltpu.VMEM((RB, DH), jnp.float32),
            pltpu.VMEM((RB, DH), jnp.float32),
            pltpu.VMEM((RB, DH), jnp.float32),
            pltpu.VMEM((EB,), jnp.float32),
            pltpu.VMEM((RT,), jnp.float32),
            pltpu.SemaphoreType.DMA,
        ],
    )
    return f(row2d, col2d, xh)


def kernel(x, adj_t):
    a = adj_t.astype(jnp.int32)
    row = jnp.concatenate([a[0], a[1]])
    col = jnp.concatenate([a[1], a[0]])
    npad = E2P - E2
    pad = N + (jnp.arange(npad, dtype=jnp.int32) % NPADROWS)
    row2d = jnp.concatenate([row, pad]).reshape(E2P // EB, EB)
    col2d = jnp.concatenate([col, pad]).reshape(E2P // EB, EB)
    xh = x.reshape(N, 2, DH).transpose(1, 0, 2)
    xh = jnp.pad(xh, ((0, 0), (0, NPADROWS), (0, 0)))
    out2, _ = _gcn(row2d, col2d, xh.reshape(2 * NT, DH))
    out2 = out2.reshape(2, NT, DH)[:, :N]
    return out2.transpose(1, 0, 2).reshape(N, D)
